# Initial kernel scaffold; baseline (speedup 1.0000x reference)
import jax
import jax.numpy as jnp
from jax.experimental import pallas as pl


def kernel(x, edge_index, batch, y, Wl0, Wr0, b0, g0, be0, Wl1, Wr1, b1, g1, be1, Wl2, Wr2, b2, g2, be2, Wl3, Wr3, b3, g3, be3, Wl4, Wr4, b4, g4, be4, Wlin, blin):
    raise NotImplementedError("write your pallas kernel here")


# speedup vs baseline: 15.8701x; 15.8701x over previous
import functools

import jax
import jax.numpy as jnp
from jax import lax
from jax.experimental import pallas as pl
from jax.experimental.pallas import tpu as pltpu
from jax.experimental.pallas import tpu_sc as plsc

_NC = 2
_NS = 16
_B = 128
_G = 4
_EG = _B * _G
_BN_EPS = 1e-5


def _make_sc_agg(R, ncols, ep, split_edges, nslabs):
    mesh = plsc.VectorSubcoreMesh(core_axis_name="c", subcore_axis_name="s")
    per_tile = ep // (_NC * _NS) if split_edges else ep // _NS
    n_groups = per_tile // _EG
    rows_per_tile = R // _NS

    def body(table_hbm, srcm_hbm, dstm_hbm, zeros_hbm, out_hbm,
             src_v, dst_v, rows_v, acc_sh, sem_i, sem_g):
        c = lax.axis_index("c")
        s = lax.axis_index("s")
        r0 = s * rows_per_tile
        pltpu.sync_copy(zeros_hbm.at[pl.ds(r0, rows_per_tile)],
                        acc_sh.at[pl.ds(r0, rows_per_tile)])
        plsc.subcore_barrier()
        slab = c if nslabs == 2 else 0
        tile_lin = c * _NS + s if split_edges else s
        base_row = tile_lin * (per_tile // _B)

        def group(g, carry):
            row0 = base_row + g * _G
            ci = pltpu.async_copy(srcm_hbm.at[slab, pl.ds(row0, _G)], src_v,
                                  sem_i)
            cd = pltpu.async_copy(dstm_hbm.at[pl.ds(row0, _G)], dst_v, sem_i)
            ci.wait()
            cd.wait()
            gs = [pltpu.async_copy(table_hbm.at[src_v.at[j]], rows_v.at[j],
                                   sem_g)
                  for j in range(_G)]
            for d in gs:
                d.wait()
            for j in range(_G):
                pltpu.sync_copy(rows_v.at[j], acc_sh.at[dst_v.at[j]],
                                add=True)
            return carry

        lax.fori_loop(0, n_groups, group, 0, unroll=False)
        plsc.subcore_barrier()
        pltpu.sync_copy(acc_sh.at[pl.ds(r0, rows_per_tile)],
                        out_hbm.at[c, pl.ds(r0, rows_per_tile)])

    return pl.kernel(
        body,
        out_type=jax.ShapeDtypeStruct((_NC, R, ncols), jnp.float32),
        mesh=mesh,
        scratch_types=[
            pltpu.VMEM((_G, _B), jnp.int32),
            pltpu.VMEM((_G, _B), jnp.int32),
            pltpu.VMEM((_G, _B, ncols), jnp.float32),
            pltpu.VMEM_SHARED((R, ncols), jnp.float32),
            pltpu.SemaphoreType.DMA,
            pltpu.SemaphoreType.DMA,
        ],
        compiler_params=pltpu.CompilerParams(use_tc_tiling_on_sc=False),
    )


def _layer0_tc(agg, x16, wl, wr, b_row, N, bn):
    grid = N // bn
    H = wl.shape[1]

    def body(agg_ref, x_ref, wl_ref, wr_ref, b_ref,
             z_ref, s1_ref, s2_ref, rec_ref):
        step = pl.program_id(0)
        a = agg_ref[0] + agg_ref[1]
        xb = x_ref[...]
        deg = a[:, 15:16] + 1.0
        rec = 1.0 / deg
        mean = (a + xb) * rec
        z = (jnp.dot(mean, wl_ref[...], preferred_element_type=jnp.float32)
             + jnp.dot(xb, wr_ref[...], preferred_element_type=jnp.float32)
             + b_ref[...])
        z_ref[...] = z
        rec_ref[...] = rec
        s1 = jnp.sum(z, axis=0, keepdims=True)
        s2 = jnp.sum(z * z, axis=0, keepdims=True)

        @pl.when(step == 0)
        def _():
            s1_ref[...] = s1
            s2_ref[...] = s2

        @pl.when(step != 0)
        def _():
            s1_ref[...] += s1
            s2_ref[...] += s2

    return pl.pallas_call(
        body,
        grid=(grid,),
        in_specs=[
            pl.BlockSpec((2, bn, 16), lambda i: (0, i, 0)),
            pl.BlockSpec((bn, 16), lambda i: (i, 0)),
            pl.BlockSpec((16, H), lambda i: (0, 0)),
            pl.BlockSpec((16, H), lambda i: (0, 0)),
            pl.BlockSpec((1, H), lambda i: (0, 0)),
        ],
        out_specs=[
            pl.BlockSpec((bn, H), lambda i: (i, 0)),
            pl.BlockSpec((1, H), lambda i: (0, 0)),
            pl.BlockSpec((1, H), lambda i: (0, 0)),
            pl.BlockSpec((bn, 1), lambda i: (i, 0)),
        ],
        out_shape=[
            jax.ShapeDtypeStruct((N, H), jnp.float32),
            jax.ShapeDtypeStruct((1, H), jnp.float32),
            jax.ShapeDtypeStruct((1, H), jnp.float32),
            jax.ShapeDtypeStruct((N, 1), jnp.float32),
        ],
    )(agg, x16, wl, wr, b_row)


def _layer_tc(agg, h2, rec, wl2, wr2, b_row, N, bn):
    grid = N // bn
    H = wl2.shape[2]

    def body(agg_ref, h_ref, rec_ref, wl_ref, wr_ref, b_ref,
             z_ref, s1_ref, s2_ref):
        step = pl.program_id(0)
        rec = rec_ref[...]
        ma = (agg_ref[0] + h_ref[0]) * rec
        mb = (agg_ref[1] + h_ref[1]) * rec
        z = (jnp.dot(ma, wl_ref[0], preferred_element_type=jnp.float32)
             + jnp.dot(mb, wl_ref[1], preferred_element_type=jnp.float32)
             + jnp.dot(h_ref[0], wr_ref[0],
                       preferred_element_type=jnp.float32)
             + jnp.dot(h_ref[1], wr_ref[1],
                       preferred_element_type=jnp.float32)
             + b_ref[...])
        z_ref[...] = z
        s1 = jnp.sum(z, axis=0, keepdims=True)
        s2 = jnp.sum(z * z, axis=0, keepdims=True)

        @pl.when(step == 0)
        def _():
            s1_ref[...] = s1
            s2_ref[...] = s2

        @pl.when(step != 0)
        def _():
            s1_ref[...] += s1
            s2_ref[...] += s2

    return pl.pallas_call(
        body,
        grid=(grid,),
        in_specs=[
            pl.BlockSpec((2, bn, 32), lambda i: (0, i, 0)),
            pl.BlockSpec((2, bn, 32), lambda i: (0, i, 0)),
            pl.BlockSpec((bn, 1), lambda i: (i, 0)),
            pl.BlockSpec((2, 32, H), lambda i: (0, 0, 0)),
            pl.BlockSpec((2, 32, H), lambda i: (0, 0, 0)),
            pl.BlockSpec((1, H), lambda i: (0, 0)),
        ],
        out_specs=[
            pl.BlockSpec((bn, H), lambda i: (i, 0)),
            pl.BlockSpec((1, H), lambda i: (0, 0)),
            pl.BlockSpec((1, H), lambda i: (0, 0)),
        ],
        out_shape=[
            jax.ShapeDtypeStruct((N, H), jnp.float32),
            jax.ShapeDtypeStruct((1, H), jnp.float32),
            jax.ShapeDtypeStruct((1, H), jnp.float32),
        ],
    )(agg, h2, rec, wl2, wr2, b_row)


def _bnrelu_tc(z, scale, shift, N, bn):
    grid = N // bn
    H = z.shape[1]

    def body(z_ref, sc_ref, sh_ref, h_ref):
        h = jnp.maximum(z_ref[...] * sc_ref[...] + sh_ref[...], 0.0)
        h_ref[0] = h[:, :32]
        h_ref[1] = h[:, 32:]

    return pl.pallas_call(
        body,
        grid=(grid,),
        in_specs=[
            pl.BlockSpec((bn, H), lambda i: (i, 0)),
            pl.BlockSpec((1, H), lambda i: (0, 0)),
            pl.BlockSpec((1, H), lambda i: (0, 0)),
        ],
        out_specs=pl.BlockSpec((2, bn, 32), lambda i: (0, i, 0)),
        out_shape=jax.ShapeDtypeStruct((2, N, 32), jnp.float32),
    )(z, scale, shift)


def _pool_tc(z, scale, shift, batchi, wlin, blin11, N, NG, bn):
    grid = N // bn
    H = z.shape[1]

    def body(z_ref, sc_ref, sh_ref, b_ref, wlin_ref, blin_ref, o_ref,
             pool_acc, cnt_acc):
        step = pl.program_id(0)
        h = jnp.maximum(z_ref[...] * sc_ref[...] + sh_ref[...], 0.0)
        gids = lax.broadcasted_iota(jnp.int32, (1, NG), 1)
        onehot = (b_ref[...] == gids).astype(jnp.float32)
        psum = lax.dot_general(onehot, h, (((0,), (0,)), ((), ())),
                               preferred_element_type=jnp.float32)
        ones = jnp.ones((bn, 1), jnp.float32)
        csum = lax.dot_general(onehot, ones, (((0,), (0,)), ((), ())),
                               preferred_element_type=jnp.float32)

        @pl.when(step == 0)
        def _():
            pool_acc[...] = psum
            cnt_acc[...] = csum

        @pl.when(step != 0)
        def _():
            pool_acc[...] += psum
            cnt_acc[...] += csum

        @pl.when(step == grid - 1)
        def _():
            pooled = pool_acc[...] / jnp.maximum(cnt_acc[...], 1.0)
            o_ref[...] = (jnp.dot(pooled, wlin_ref[...],
                                  preferred_element_type=jnp.float32)
                          + blin_ref[...])

    return pl.pallas_call(
        body,
        grid=(grid,),
        in_specs=[
            pl.BlockSpec((bn, H), lambda i: (i, 0)),
            pl.BlockSpec((1, H), lambda i: (0, 0)),
            pl.BlockSpec((1, H), lambda i: (0, 0)),
            pl.BlockSpec((bn, 1), lambda i: (i, 0)),
            pl.BlockSpec((H, 1), lambda i: (0, 0)),
            pl.BlockSpec((1, 1), lambda i: (0, 0)),
        ],
        out_specs=pl.BlockSpec((NG, 1), lambda i: (0, 0)),
        out_shape=jax.ShapeDtypeStruct((NG, 1), jnp.float32),
        scratch_shapes=[
            pltpu.VMEM((NG, H), jnp.float32),
            pltpu.VMEM((NG, 1), jnp.float32),
        ],
    )(z, scale, shift, batchi, wlin, blin11)


def _bn_coeffs(s1, s2, g, be, N):
    mu = s1 / N
    var = s2 / N - mu * mu
    scale = g.reshape(1, -1) / jnp.sqrt(var + _BN_EPS)
    shift = be.reshape(1, -1) - mu * scale
    return scale, shift


def kernel(x, edge_index, batch, y,
           Wl0, Wr0, b0, g0, be0,
           Wl1, Wr1, b1, g1, be1,
           Wl2, Wr2, b2, g2, be2,
           Wl3, Wr3, b3, g3, be3,
           Wl4, Wr4, b4, g4, be4,
           Wlin, blin):
    N, DIN = x.shape
    E = edge_index.shape[1]
    H = Wl0.shape[1]
    NG = y.shape[0]
    NL = 5
    bn = 1000

    chunk = _NC * _NS * _EG
    EP = -(-E // chunk) * chunk
    R = -(-(N + 48) // _NS) * _NS
    P = EP - E

    src = edge_index[0]
    dst = edge_index[1]
    pad_i = jnp.arange(P, dtype=jnp.int32)
    src_p = jnp.concatenate([src, pad_i % jnp.int32(N)])
    dst_p = jnp.concatenate([dst, jnp.int32(N) + pad_i % jnp.int32(R - N)])
    dstm = dst_p.reshape(EP // _B, _B)
    srcm1 = src_p.reshape(1, EP // _B, _B)
    srcm2 = jnp.stack([src_p, src_p + jnp.int32(N)]).reshape(2, EP // _B, _B)

    zeros16 = jnp.zeros((R, 16), jnp.float32)
    zeros32 = jnp.zeros((R, 32), jnp.float32)
    x16 = jnp.concatenate([x, jnp.ones((N, 1), jnp.float32)], axis=1)

    zrow = jnp.zeros((1, H), jnp.float32)
    wl0p = jnp.concatenate([Wl0, zrow], axis=0)
    wr0p = jnp.concatenate([Wr0, zrow], axis=0)

    agg16 = _make_sc_agg(R, 16, EP, split_edges=True, nslabs=1)
    agg32 = _make_sc_agg(R, 32, EP, split_edges=False, nslabs=2)

    a0 = agg16(x16, srcm1, dstm, zeros16)
    z, s1, s2, rec = _layer0_tc(a0, x16, wl0p, wr0p, b0.reshape(1, H), N, bn)
    scale, shift = _bn_coeffs(s1, s2, g0, be0, N)
    h2 = _bnrelu_tc(z, scale, shift, N, bn)

    layers = [(Wl1, Wr1, b1, g1, be1), (Wl2, Wr2, b2, g2, be2),
              (Wl3, Wr3, b3, g3, be3), (Wl4, Wr4, b4, g4, be4)]
    for i, (wl, wr, b, g, be) in enumerate(layers):
        table = h2.reshape(2 * N, 32)
        agg = agg32(table, srcm2, dstm, zeros32)
        z, s1, s2 = _layer_tc(agg, h2, rec, wl.reshape(2, 32, H),
                              wr.reshape(2, 32, H), b.reshape(1, H), N, bn)
        scale, shift = _bn_coeffs(s1, s2, g, be, N)
        if i < len(layers) - 1:
            h2 = _bnrelu_tc(z, scale, shift, N, bn)

    batchi = batch.reshape(N, 1)
    out = _pool_tc(z, scale, shift, batchi, Wlin, blin.reshape(1, 1),
                   N, NG, bn)
    return out


# speedup vs baseline: 17.8958x; 1.1276x over previous
import functools

import jax
import jax.numpy as jnp
from jax import lax
from jax.experimental import pallas as pl
from jax.experimental.pallas import tpu as pltpu
from jax.experimental.pallas import tpu_sc as plsc

_NC = 2
_NS = 16
_B = 128
_G = 2
_KG = 4
_EG = _B * _G * _KG
_BN_EPS = 1e-5


def _make_sc_agg(R, ncols, ep, split_edges, nslabs):
    mesh = plsc.VectorSubcoreMesh(core_axis_name="c", subcore_axis_name="s")
    per_tile = ep // (_NC * _NS) if split_edges else ep // _NS
    n_outer = per_tile // _EG
    idx_rows = _KG * _G
    rows_per_tile = R // _NS

    def body(table_hbm, srcm_hbm, dstm_hbm, zeros_hbm, out_hbm,
             src_v, dst_v, rows0, rows1, acc_sh,
             sem_g0, sem_g1, sem_a0, sem_a1):
        c = lax.axis_index("c")
        s = lax.axis_index("s")
        r0 = s * rows_per_tile
        pltpu.sync_copy(zeros_hbm.at[pl.ds(r0, rows_per_tile)],
                        acc_sh.at[pl.ds(r0, rows_per_tile)])
        plsc.subcore_barrier()
        slab = c if nslabs == 2 else 0
        tile_lin = c * _NS + s if split_edges else s
        base_row = tile_lin * (per_tile // _B)
        rows = [rows0, rows1]
        sem_g = [sem_g0, sem_g1]
        sem_a = [sem_a0, sem_a1]

        def outer(t, carry):
            row0 = base_row + t * idx_rows
            pltpu.sync_copy(srcm_hbm.at[slab, pl.ds(row0, idx_rows)], src_v)
            pltpu.sync_copy(dstm_hbm.at[pl.ds(row0, idx_rows)], dst_v)

            def fire_gather(k):
                buf = k % 2
                return [pltpu.async_copy(table_hbm.at[src_v.at[k * _G + j]],
                                         rows[buf].at[j], sem_g[buf])
                        for j in range(_G)]

            gath = [None, None]
            adds = [None, None]
            gath[0] = fire_gather(0)
            for k in range(_KG):
                buf = k % 2
                if k + 1 < _KG:
                    nxt = (k + 1) % 2
                    if adds[nxt] is not None:
                        for d in adds[nxt]:
                            d.wait()
                        adds[nxt] = None
                    gath[nxt] = fire_gather(k + 1)
                for d in gath[buf]:
                    d.wait()
                adds[buf] = [
                    pltpu.async_copy(rows[buf].at[j],
                                     acc_sh.at[dst_v.at[k * _G + j]],
                                     sem_a[buf], add=True)
                    for j in range(_G)
                ]
            for b in range(2):
                if adds[b] is not None:
                    for d in adds[b]:
                        d.wait()
            return carry

        lax.fori_loop(0, n_outer, outer, 0, unroll=False)
        plsc.subcore_barrier()
        pltpu.sync_copy(acc_sh.at[pl.ds(r0, rows_per_tile)],
                        out_hbm.at[c, pl.ds(r0, rows_per_tile)])

    return pl.kernel(
        body,
        out_type=jax.ShapeDtypeStruct((_NC, R, ncols), jnp.float32),
        mesh=mesh,
        scratch_types=[
            pltpu.VMEM((idx_rows, _B), jnp.int32),
            pltpu.VMEM((idx_rows, _B), jnp.int32),
            pltpu.VMEM((_G, _B, ncols), jnp.float32),
            pltpu.VMEM((_G, _B, ncols), jnp.float32),
            pltpu.VMEM_SHARED((R, ncols), jnp.float32),
            pltpu.SemaphoreType.DMA,
            pltpu.SemaphoreType.DMA,
            pltpu.SemaphoreType.DMA,
            pltpu.SemaphoreType.DMA,
        ],
        compiler_params=pltpu.CompilerParams(use_tc_tiling_on_sc=False),
    )


def _layer0_tc(agg, x16, wl, wr, b_row, N, bn):
    grid = N // bn
    H = wl.shape[1]

    def body(agg_ref, x_ref, wl_ref, wr_ref, b_ref,
             z_ref, s1_ref, s2_ref, rec_ref):
        step = pl.program_id(0)
        a = agg_ref[0] + agg_ref[1]
        xb = x_ref[...]
        deg = a[:, 15:16] + 1.0
        rec = 1.0 / deg
        mean = (a + xb) * rec
        z = (jnp.dot(mean, wl_ref[...], preferred_element_type=jnp.float32)
             + jnp.dot(xb, wr_ref[...], preferred_element_type=jnp.float32)
             + b_ref[...])
        z_ref[...] = z
        rec_ref[...] = rec
        s1 = jnp.sum(z, axis=0, keepdims=True)
        s2 = jnp.sum(z * z, axis=0, keepdims=True)

        @pl.when(step == 0)
        def _():
            s1_ref[...] = s1
            s2_ref[...] = s2

        @pl.when(step != 0)
        def _():
            s1_ref[...] += s1
            s2_ref[...] += s2

    return pl.pallas_call(
        body,
        grid=(grid,),
        in_specs=[
            pl.BlockSpec((2, bn, 16), lambda i: (0, i, 0)),
            pl.BlockSpec((bn, 16), lambda i: (i, 0)),
            pl.BlockSpec((16, H), lambda i: (0, 0)),
            pl.BlockSpec((16, H), lambda i: (0, 0)),
            pl.BlockSpec((1, H), lambda i: (0, 0)),
        ],
        out_specs=[
            pl.BlockSpec((bn, H), lambda i: (i, 0)),
            pl.BlockSpec((1, H), lambda i: (0, 0)),
            pl.BlockSpec((1, H), lambda i: (0, 0)),
            pl.BlockSpec((bn, 1), lambda i: (i, 0)),
        ],
        out_shape=[
            jax.ShapeDtypeStruct((N, H), jnp.float32),
            jax.ShapeDtypeStruct((1, H), jnp.float32),
            jax.ShapeDtypeStruct((1, H), jnp.float32),
            jax.ShapeDtypeStruct((N, 1), jnp.float32),
        ],
    )(agg, x16, wl, wr, b_row)


def _layer_tc(agg, h2, rec, wl2, wr2, b_row, N, bn):
    grid = N // bn
    H = wl2.shape[2]

    def body(agg_ref, h_ref, rec_ref, wl_ref, wr_ref, b_ref,
             z_ref, s1_ref, s2_ref):
        step = pl.program_id(0)
        rec = rec_ref[...]
        ma = (agg_ref[0] + h_ref[0]) * rec
        mb = (agg_ref[1] + h_ref[1]) * rec
        z = (jnp.dot(ma, wl_ref[0], preferred_element_type=jnp.float32)
             + jnp.dot(mb, wl_ref[1], preferred_element_type=jnp.float32)
             + jnp.dot(h_ref[0], wr_ref[0],
                       preferred_element_type=jnp.float32)
             + jnp.dot(h_ref[1], wr_ref[1],
                       preferred_element_type=jnp.float32)
             + b_ref[...])
        z_ref[...] = z
        s1 = jnp.sum(z, axis=0, keepdims=True)
        s2 = jnp.sum(z * z, axis=0, keepdims=True)

        @pl.when(step == 0)
        def _():
            s1_ref[...] = s1
            s2_ref[...] = s2

        @pl.when(step != 0)
        def _():
            s1_ref[...] += s1
            s2_ref[...] += s2

    return pl.pallas_call(
        body,
        grid=(grid,),
        in_specs=[
            pl.BlockSpec((2, bn, 32), lambda i: (0, i, 0)),
            pl.BlockSpec((2, bn, 32), lambda i: (0, i, 0)),
            pl.BlockSpec((bn, 1), lambda i: (i, 0)),
            pl.BlockSpec((2, 32, H), lambda i: (0, 0, 0)),
            pl.BlockSpec((2, 32, H), lambda i: (0, 0, 0)),
            pl.BlockSpec((1, H), lambda i: (0, 0)),
        ],
        out_specs=[
            pl.BlockSpec((bn, H), lambda i: (i, 0)),
            pl.BlockSpec((1, H), lambda i: (0, 0)),
            pl.BlockSpec((1, H), lambda i: (0, 0)),
        ],
        out_shape=[
            jax.ShapeDtypeStruct((N, H), jnp.float32),
            jax.ShapeDtypeStruct((1, H), jnp.float32),
            jax.ShapeDtypeStruct((1, H), jnp.float32),
        ],
    )(agg, h2, rec, wl2, wr2, b_row)


def _bnrelu_tc(z, scale, shift, N, bn):
    grid = N // bn
    H = z.shape[1]

    def body(z_ref, sc_ref, sh_ref, h_ref):
        h = jnp.maximum(z_ref[...] * sc_ref[...] + sh_ref[...], 0.0)
        h_ref[0] = h[:, :32]
        h_ref[1] = h[:, 32:]

    return pl.pallas_call(
        body,
        grid=(grid,),
        in_specs=[
            pl.BlockSpec((bn, H), lambda i: (i, 0)),
            pl.BlockSpec((1, H), lambda i: (0, 0)),
            pl.BlockSpec((1, H), lambda i: (0, 0)),
        ],
        out_specs=pl.BlockSpec((2, bn, 32), lambda i: (0, i, 0)),
        out_shape=jax.ShapeDtypeStruct((2, N, 32), jnp.float32),
    )(z, scale, shift)


def _pool_tc(z, scale, shift, batchi, wlin, blin11, N, NG, bn):
    grid = N // bn
    H = z.shape[1]

    def body(z_ref, sc_ref, sh_ref, b_ref, wlin_ref, blin_ref, o_ref,
             pool_acc, cnt_acc):
        step = pl.program_id(0)
        h = jnp.maximum(z_ref[...] * sc_ref[...] + sh_ref[...], 0.0)
        gids = lax.broadcasted_iota(jnp.int32, (1, NG), 1)
        onehot = (b_ref[...] == gids).astype(jnp.float32)
        psum = lax.dot_general(onehot, h, (((0,), (0,)), ((), ())),
                               preferred_element_type=jnp.float32)
        ones = jnp.ones((bn, 1), jnp.float32)
        csum = lax.dot_general(onehot, ones, (((0,), (0,)), ((), ())),
                               preferred_element_type=jnp.float32)

        @pl.when(step == 0)
        def _():
            pool_acc[...] = psum
            cnt_acc[...] = csum

        @pl.when(step != 0)
        def _():
            pool_acc[...] += psum
            cnt_acc[...] += csum

        @pl.when(step == grid - 1)
        def _():
            pooled = pool_acc[...] / jnp.maximum(cnt_acc[...], 1.0)
            o_ref[...] = (jnp.dot(pooled, wlin_ref[...],
                                  preferred_element_type=jnp.float32)
                          + blin_ref[...])

    return pl.pallas_call(
        body,
        grid=(grid,),
        in_specs=[
            pl.BlockSpec((bn, H), lambda i: (i, 0)),
            pl.BlockSpec((1, H), lambda i: (0, 0)),
            pl.BlockSpec((1, H), lambda i: (0, 0)),
            pl.BlockSpec((bn, 1), lambda i: (i, 0)),
            pl.BlockSpec((H, 1), lambda i: (0, 0)),
            pl.BlockSpec((1, 1), lambda i: (0, 0)),
        ],
        out_specs=pl.BlockSpec((NG, 1), lambda i: (0, 0)),
        out_shape=jax.ShapeDtypeStruct((NG, 1), jnp.float32),
        scratch_shapes=[
            pltpu.VMEM((NG, H), jnp.float32),
            pltpu.VMEM((NG, 1), jnp.float32),
        ],
    )(z, scale, shift, batchi, wlin, blin11)


def _bn_coeffs(s1, s2, g, be, N):
    mu = s1 / N
    var = s2 / N - mu * mu
    scale = g.reshape(1, -1) / jnp.sqrt(var + _BN_EPS)
    shift = be.reshape(1, -1) - mu * scale
    return scale, shift


def kernel(x, edge_index, batch, y,
           Wl0, Wr0, b0, g0, be0,
           Wl1, Wr1, b1, g1, be1,
           Wl2, Wr2, b2, g2, be2,
           Wl3, Wr3, b3, g3, be3,
           Wl4, Wr4, b4, g4, be4,
           Wlin, blin):
    N, DIN = x.shape
    E = edge_index.shape[1]
    H = Wl0.shape[1]
    NG = y.shape[0]
    NL = 5
    bn = 1000

    chunk = _NC * _NS * _EG
    EP = -(-E // chunk) * chunk
    R = -(-(N + 48) // _NS) * _NS
    P = EP - E

    src = edge_index[0]
    dst = edge_index[1]
    pad_i = jnp.arange(P, dtype=jnp.int32)
    src_p = jnp.concatenate([src, pad_i % jnp.int32(N)])
    dst_p = jnp.concatenate([dst, jnp.int32(N) + pad_i % jnp.int32(R - N)])
    dstm = dst_p.reshape(EP // _B, _B)
    srcm1 = src_p.reshape(1, EP // _B, _B)
    srcm2 = jnp.stack([src_p, src_p + jnp.int32(N)]).reshape(2, EP // _B, _B)

    zeros16 = jnp.zeros((R, 16), jnp.float32)
    zeros32 = jnp.zeros((R, 32), jnp.float32)
    x16 = jnp.concatenate([x, jnp.ones((N, 1), jnp.float32)], axis=1)

    zrow = jnp.zeros((1, H), jnp.float32)
    wl0p = jnp.concatenate([Wl0, zrow], axis=0)
    wr0p = jnp.concatenate([Wr0, zrow], axis=0)

    agg16 = _make_sc_agg(R, 16, EP, split_edges=True, nslabs=1)
    agg32 = _make_sc_agg(R, 32, EP, split_edges=False, nslabs=2)

    a0 = agg16(x16, srcm1, dstm, zeros16)
    z, s1, s2, rec = _layer0_tc(a0, x16, wl0p, wr0p, b0.reshape(1, H), N, bn)
    scale, shift = _bn_coeffs(s1, s2, g0, be0, N)
    h2 = _bnrelu_tc(z, scale, shift, N, bn)

    layers = [(Wl1, Wr1, b1, g1, be1), (Wl2, Wr2, b2, g2, be2),
              (Wl3, Wr3, b3, g3, be3), (Wl4, Wr4, b4, g4, be4)]
    for i, (wl, wr, b, g, be) in enumerate(layers):
        table = h2.reshape(2 * N, 32)
        agg = agg32(table, srcm2, dstm, zeros32)
        z, s1, s2 = _layer_tc(agg, h2, rec, wl.reshape(2, 32, H),
                              wr.reshape(2, 32, H), b.reshape(1, H), N, bn)
        scale, shift = _bn_coeffs(s1, s2, g, be, N)
        if i < len(layers) - 1:
            h2 = _bnrelu_tc(z, scale, shift, N, bn)

    batchi = batch.reshape(N, 1)
    out = _pool_tc(z, scale, shift, batchi, Wlin, blin.reshape(1, 1),
                   N, NG, bn)
    return out


# speedup vs baseline: 21.1488x; 1.1818x over previous
import functools

import jax
import jax.numpy as jnp
from jax import lax
from jax.experimental import pallas as pl
from jax.experimental.pallas import tpu as pltpu
from jax.experimental.pallas import tpu_sc as plsc

_NC = 2
_NS = 16
_B = 128
_G = 2
_KG = 8
_D = 3
_EG = _B * _G * _KG
_BN_EPS = 1e-5


def _make_sc_agg(R, ncols, ep, split_edges, nslabs):
    mesh = plsc.VectorSubcoreMesh(core_axis_name="c", subcore_axis_name="s")
    per_tile = ep // (_NC * _NS) if split_edges else ep // _NS
    n_outer = per_tile // _EG
    idx_rows = _KG * _G
    rows_per_tile = R // _NS

    def body(table_hbm, srcm_hbm, dstm_hbm, zeros_hbm, out_hbm,
             src_v, dst_v, rows0, rows1, rows2, acc_sh,
             sem_g0, sem_g1, sem_g2, sem_a0, sem_a1, sem_a2):
        c = lax.axis_index("c")
        s = lax.axis_index("s")
        r0 = s * rows_per_tile
        pltpu.sync_copy(zeros_hbm.at[pl.ds(r0, rows_per_tile)],
                        acc_sh.at[pl.ds(r0, rows_per_tile)])
        plsc.subcore_barrier()
        slab = c if nslabs == 2 else 0
        tile_lin = c * _NS + s if split_edges else s
        base_row = tile_lin * (per_tile // _B)
        rows = [rows0, rows1, rows2]
        sem_g = [sem_g0, sem_g1, sem_g2]
        sem_a = [sem_a0, sem_a1, sem_a2]

        def outer(t, carry):
            row0 = base_row + t * idx_rows
            pltpu.sync_copy(srcm_hbm.at[slab, pl.ds(row0, idx_rows)], src_v)
            pltpu.sync_copy(dstm_hbm.at[pl.ds(row0, idx_rows)], dst_v)

            def fire_gather(k):
                buf = k % _D
                return [pltpu.async_copy(table_hbm.at[src_v.at[k * _G + j]],
                                         rows[buf].at[j], sem_g[buf])
                        for j in range(_G)]

            def fire_adds(k):
                buf = k % _D
                return [
                    pltpu.async_copy(rows[buf].at[j],
                                     acc_sh.at[dst_v.at[k * _G + j]],
                                     sem_a[buf], add=True)
                    for j in range(_G)
                ]

            gath = [None] * _D
            adds = [None] * _D
            for k in range(_KG + _D - 1):
                if k < _KG:
                    buf = k % _D
                    if adds[buf] is not None:
                        for d in adds[buf]:
                            d.wait()
                        adds[buf] = None
                    gath[buf] = fire_gather(k)
                kk = k - (_D - 1)
                if kk >= 0:
                    buf = kk % _D
                    for d in gath[buf]:
                        d.wait()
                    adds[buf] = fire_adds(kk)
            for b in range(_D):
                if adds[b] is not None:
                    for d in adds[b]:
                        d.wait()
            return carry

        lax.fori_loop(0, n_outer, outer, 0, unroll=False)
        plsc.subcore_barrier()
        pltpu.sync_copy(acc_sh.at[pl.ds(r0, rows_per_tile)],
                        out_hbm.at[c, pl.ds(r0, rows_per_tile)])

    return pl.kernel(
        body,
        out_type=jax.ShapeDtypeStruct((_NC, R, ncols), jnp.float32),
        mesh=mesh,
        scratch_types=[
            pltpu.VMEM((idx_rows, _B), jnp.int32),
            pltpu.VMEM((idx_rows, _B), jnp.int32),
            pltpu.VMEM((_G, _B, ncols), jnp.float32),
            pltpu.VMEM((_G, _B, ncols), jnp.float32),
            pltpu.VMEM((_G, _B, ncols), jnp.float32),
            pltpu.VMEM_SHARED((R, ncols), jnp.float32),
            pltpu.SemaphoreType.DMA,
            pltpu.SemaphoreType.DMA,
            pltpu.SemaphoreType.DMA,
            pltpu.SemaphoreType.DMA,
            pltpu.SemaphoreType.DMA,
            pltpu.SemaphoreType.DMA,
        ],
        compiler_params=pltpu.CompilerParams(use_tc_tiling_on_sc=False),
    )


def _layer0_tc(agg, x16, wl, wr, b_row, N, bn):
    grid = N // bn
    H = wl.shape[1]

    def body(agg_ref, x_ref, wl_ref, wr_ref, b_ref,
             z_ref, s1_ref, s2_ref, rec_ref):
        step = pl.program_id(0)
        a = agg_ref[0] + agg_ref[1]
        xb = x_ref[...]
        deg = a[:, 15:16] + 1.0
        rec = 1.0 / deg
        mean = (a + xb) * rec
        z = (jnp.dot(mean, wl_ref[...], preferred_element_type=jnp.float32)
             + jnp.dot(xb, wr_ref[...], preferred_element_type=jnp.float32)
             + b_ref[...])
        z_ref[...] = z
        rec_ref[...] = rec
        s1 = jnp.sum(z, axis=0, keepdims=True)
        s2 = jnp.sum(z * z, axis=0, keepdims=True)

        @pl.when(step == 0)
        def _():
            s1_ref[...] = s1
            s2_ref[...] = s2

        @pl.when(step != 0)
        def _():
            s1_ref[...] += s1
            s2_ref[...] += s2

    return pl.pallas_call(
        body,
        grid=(grid,),
        in_specs=[
            pl.BlockSpec((2, bn, 16), lambda i: (0, i, 0)),
            pl.BlockSpec((bn, 16), lambda i: (i, 0)),
            pl.BlockSpec((16, H), lambda i: (0, 0)),
            pl.BlockSpec((16, H), lambda i: (0, 0)),
            pl.BlockSpec((1, H), lambda i: (0, 0)),
        ],
        out_specs=[
            pl.BlockSpec((bn, H), lambda i: (i, 0)),
            pl.BlockSpec((1, H), lambda i: (0, 0)),
            pl.BlockSpec((1, H), lambda i: (0, 0)),
            pl.BlockSpec((bn, 1), lambda i: (i, 0)),
        ],
        out_shape=[
            jax.ShapeDtypeStruct((N, H), jnp.float32),
            jax.ShapeDtypeStruct((1, H), jnp.float32),
            jax.ShapeDtypeStruct((1, H), jnp.float32),
            jax.ShapeDtypeStruct((N, 1), jnp.float32),
        ],
    )(agg, x16, wl, wr, b_row)


def _layer_tc(agg, h2, rec, wl2, wr2, b_row, N, bn):
    grid = N // bn
    H = wl2.shape[2]

    def body(agg_ref, h_ref, rec_ref, wl_ref, wr_ref, b_ref,
             z_ref, s1_ref, s2_ref):
        step = pl.program_id(0)
        rec = rec_ref[...]
        ma = (agg_ref[0] + h_ref[0]) * rec
        mb = (agg_ref[1] + h_ref[1]) * rec
        z = (jnp.dot(ma, wl_ref[0], preferred_element_type=jnp.float32)
             + jnp.dot(mb, wl_ref[1], preferred_element_type=jnp.float32)
             + jnp.dot(h_ref[0], wr_ref[0],
                       preferred_element_type=jnp.float32)
             + jnp.dot(h_ref[1], wr_ref[1],
                       preferred_element_type=jnp.float32)
             + b_ref[...])
        z_ref[...] = z
        s1 = jnp.sum(z, axis=0, keepdims=True)
        s2 = jnp.sum(z * z, axis=0, keepdims=True)

        @pl.when(step == 0)
        def _():
            s1_ref[...] = s1
            s2_ref[...] = s2

        @pl.when(step != 0)
        def _():
            s1_ref[...] += s1
            s2_ref[...] += s2

    return pl.pallas_call(
        body,
        grid=(grid,),
        in_specs=[
            pl.BlockSpec((2, bn, 32), lambda i: (0, i, 0)),
            pl.BlockSpec((2, bn, 32), lambda i: (0, i, 0)),
            pl.BlockSpec((bn, 1), lambda i: (i, 0)),
            pl.BlockSpec((2, 32, H), lambda i: (0, 0, 0)),
            pl.BlockSpec((2, 32, H), lambda i: (0, 0, 0)),
            pl.BlockSpec((1, H), lambda i: (0, 0)),
        ],
        out_specs=[
            pl.BlockSpec((bn, H), lambda i: (i, 0)),
            pl.BlockSpec((1, H), lambda i: (0, 0)),
            pl.BlockSpec((1, H), lambda i: (0, 0)),
        ],
        out_shape=[
            jax.ShapeDtypeStruct((N, H), jnp.float32),
            jax.ShapeDtypeStruct((1, H), jnp.float32),
            jax.ShapeDtypeStruct((1, H), jnp.float32),
        ],
    )(agg, h2, rec, wl2, wr2, b_row)


def _bnrelu_tc(z, scale, shift, N, bn):
    grid = N // bn
    H = z.shape[1]

    def body(z_ref, sc_ref, sh_ref, h_ref):
        h = jnp.maximum(z_ref[...] * sc_ref[...] + sh_ref[...], 0.0)
        h_ref[0] = h[:, :32]
        h_ref[1] = h[:, 32:]

    return pl.pallas_call(
        body,
        grid=(grid,),
        in_specs=[
            pl.BlockSpec((bn, H), lambda i: (i, 0)),
            pl.BlockSpec((1, H), lambda i: (0, 0)),
            pl.BlockSpec((1, H), lambda i: (0, 0)),
        ],
        out_specs=pl.BlockSpec((2, bn, 32), lambda i: (0, i, 0)),
        out_shape=jax.ShapeDtypeStruct((2, N, 32), jnp.float32),
    )(z, scale, shift)


def _pool_tc(z, scale, shift, batchi, wlin, blin11, N, NG, bn):
    grid = N // bn
    H = z.shape[1]

    def body(z_ref, sc_ref, sh_ref, b_ref, wlin_ref, blin_ref, o_ref,
             pool_acc, cnt_acc):
        step = pl.program_id(0)
        h = jnp.maximum(z_ref[...] * sc_ref[...] + sh_ref[...], 0.0)
        gids = lax.broadcasted_iota(jnp.int32, (1, NG), 1)
        onehot = (b_ref[...] == gids).astype(jnp.float32)
        psum = lax.dot_general(onehot, h, (((0,), (0,)), ((), ())),
                               preferred_element_type=jnp.float32)
        ones = jnp.ones((bn, 1), jnp.float32)
        csum = lax.dot_general(onehot, ones, (((0,), (0,)), ((), ())),
                               preferred_element_type=jnp.float32)

        @pl.when(step == 0)
        def _():
            pool_acc[...] = psum
            cnt_acc[...] = csum

        @pl.when(step != 0)
        def _():
            pool_acc[...] += psum
            cnt_acc[...] += csum

        @pl.when(step == grid - 1)
        def _():
            pooled = pool_acc[...] / jnp.maximum(cnt_acc[...], 1.0)
            o_ref[...] = (jnp.dot(pooled, wlin_ref[...],
                                  preferred_element_type=jnp.float32)
                          + blin_ref[...])

    return pl.pallas_call(
        body,
        grid=(grid,),
        in_specs=[
            pl.BlockSpec((bn, H), lambda i: (i, 0)),
            pl.BlockSpec((1, H), lambda i: (0, 0)),
            pl.BlockSpec((1, H), lambda i: (0, 0)),
            pl.BlockSpec((bn, 1), lambda i: (i, 0)),
            pl.BlockSpec((H, 1), lambda i: (0, 0)),
            pl.BlockSpec((1, 1), lambda i: (0, 0)),
        ],
        out_specs=pl.BlockSpec((NG, 1), lambda i: (0, 0)),
        out_shape=jax.ShapeDtypeStruct((NG, 1), jnp.float32),
        scratch_shapes=[
            pltpu.VMEM((NG, H), jnp.float32),
            pltpu.VMEM((NG, 1), jnp.float32),
        ],
    )(z, scale, shift, batchi, wlin, blin11)


def _bn_coeffs(s1, s2, g, be, N):
    mu = s1 / N
    var = s2 / N - mu * mu
    scale = g.reshape(1, -1) / jnp.sqrt(var + _BN_EPS)
    shift = be.reshape(1, -1) - mu * scale
    return scale, shift


def kernel(x, edge_index, batch, y,
           Wl0, Wr0, b0, g0, be0,
           Wl1, Wr1, b1, g1, be1,
           Wl2, Wr2, b2, g2, be2,
           Wl3, Wr3, b3, g3, be3,
           Wl4, Wr4, b4, g4, be4,
           Wlin, blin):
    N, DIN = x.shape
    E = edge_index.shape[1]
    H = Wl0.shape[1]
    NG = y.shape[0]
    NL = 5
    bn = 1000

    chunk = _NC * _NS * _EG
    EP = -(-E // chunk) * chunk
    R = -(-(N + 48) // _NS) * _NS
    P = EP - E

    src = edge_index[0]
    dst = edge_index[1]
    pad_i = jnp.arange(P, dtype=jnp.int32)
    src_p = jnp.concatenate([src, pad_i % jnp.int32(N)])
    dst_p = jnp.concatenate([dst, jnp.int32(N) + pad_i % jnp.int32(R - N)])
    dstm = dst_p.reshape(EP // _B, _B)
    srcm1 = src_p.reshape(1, EP // _B, _B)
    srcm2 = jnp.stack([src_p, src_p + jnp.int32(N)]).reshape(2, EP // _B, _B)

    zeros16 = jnp.zeros((R, 16), jnp.float32)
    zeros32 = jnp.zeros((R, 32), jnp.float32)
    x16 = jnp.concatenate([x, jnp.ones((N, 1), jnp.float32)], axis=1)

    zrow = jnp.zeros((1, H), jnp.float32)
    wl0p = jnp.concatenate([Wl0, zrow], axis=0)
    wr0p = jnp.concatenate([Wr0, zrow], axis=0)

    agg16 = _make_sc_agg(R, 16, EP, split_edges=True, nslabs=1)
    agg32 = _make_sc_agg(R, 32, EP, split_edges=False, nslabs=2)

    a0 = agg16(x16, srcm1, dstm, zeros16)
    z, s1, s2, rec = _layer0_tc(a0, x16, wl0p, wr0p, b0.reshape(1, H), N, bn)
    scale, shift = _bn_coeffs(s1, s2, g0, be0, N)
    h2 = _bnrelu_tc(z, scale, shift, N, bn)

    layers = [(Wl1, Wr1, b1, g1, be1), (Wl2, Wr2, b2, g2, be2),
              (Wl3, Wr3, b3, g3, be3), (Wl4, Wr4, b4, g4, be4)]
    for i, (wl, wr, b, g, be) in enumerate(layers):
        table = h2.reshape(2 * N, 32)
        agg = agg32(table, srcm2, dstm, zeros32)
        z, s1, s2 = _layer_tc(agg, h2, rec, wl.reshape(2, 32, H),
                              wr.reshape(2, 32, H), b.reshape(1, H), N, bn)
        scale, shift = _bn_coeffs(s1, s2, g, be, N)
        if i < len(layers) - 1:
            h2 = _bnrelu_tc(z, scale, shift, N, bn)

    batchi = batch.reshape(N, 1)
    out = _pool_tc(z, scale, shift, batchi, Wlin, blin.reshape(1, 1),
                   N, NG, bn)
    return out


# speedup vs baseline: 21.8213x; 1.0318x over previous
import functools

import jax
import jax.numpy as jnp
from jax import lax
from jax.experimental import pallas as pl
from jax.experimental.pallas import tpu as pltpu
from jax.experimental.pallas import tpu_sc as plsc

_NC = 2
_NS = 16
_B = 128
_G = 2
_KG = 8
_D = 3
_EG = _B * _G * _KG
_BN_EPS = 1e-5


def _make_sc_agg(R, ncols, ep, split_edges, nslabs):
    mesh = plsc.VectorSubcoreMesh(core_axis_name="c", subcore_axis_name="s")
    per_tile = ep // (_NC * _NS) if split_edges else ep // _NS
    n_outer = per_tile // _EG
    idx_rows = _KG * _G
    rows_per_tile = R // _NS

    def body(table_hbm, srcm_hbm, dstm_hbm, zeros_hbm, out_hbm,
             src_v, dst_v, rows0, rows1, rows2, acc_sh,
             sem_g0, sem_g1, sem_g2, sem_a0, sem_a1, sem_a2):
        c = lax.axis_index("c")
        s = lax.axis_index("s")
        r0 = s * rows_per_tile
        pltpu.sync_copy(zeros_hbm.at[pl.ds(r0, rows_per_tile)],
                        acc_sh.at[pl.ds(r0, rows_per_tile)])
        plsc.subcore_barrier()
        slab = c if nslabs == 2 else 0
        tile_lin = c * _NS + s if split_edges else s
        base_row = tile_lin * (per_tile // _B)
        rows = [rows0, rows1, rows2]
        sem_g = [sem_g0, sem_g1, sem_g2]
        sem_a = [sem_a0, sem_a1, sem_a2]

        def outer(t, carry):
            row0 = base_row + t * idx_rows
            pltpu.sync_copy(srcm_hbm.at[slab, pl.ds(row0, idx_rows)], src_v)
            pltpu.sync_copy(dstm_hbm.at[pl.ds(row0, idx_rows)], dst_v)

            def fire_gather(k):
                buf = k % _D
                return [pltpu.async_copy(table_hbm.at[src_v.at[k * _G + j]],
                                         rows[buf].at[j], sem_g[buf])
                        for j in range(_G)]

            def fire_adds(k):
                buf = k % _D
                return [
                    pltpu.async_copy(rows[buf].at[j],
                                     acc_sh.at[dst_v.at[k * _G + j]],
                                     sem_a[buf], add=True)
                    for j in range(_G)
                ]

            gath = [None] * _D
            adds = [None] * _D
            for k in range(_KG + _D - 1):
                if k < _KG:
                    buf = k % _D
                    if adds[buf] is not None:
                        for d in adds[buf]:
                            d.wait()
                        adds[buf] = None
                    gath[buf] = fire_gather(k)
                kk = k - (_D - 1)
                if kk >= 0:
                    buf = kk % _D
                    for d in gath[buf]:
                        d.wait()
                    adds[buf] = fire_adds(kk)
            for b in range(_D):
                if adds[b] is not None:
                    for d in adds[b]:
                        d.wait()
            return carry

        lax.fori_loop(0, n_outer, outer, 0, unroll=False)
        plsc.subcore_barrier()
        pltpu.sync_copy(acc_sh.at[pl.ds(r0, rows_per_tile)],
                        out_hbm.at[c, pl.ds(r0, rows_per_tile)])

    return pl.kernel(
        body,
        out_type=jax.ShapeDtypeStruct((_NC, R, ncols), jnp.float32),
        mesh=mesh,
        scratch_types=[
            pltpu.VMEM((idx_rows, _B), jnp.int32),
            pltpu.VMEM((idx_rows, _B), jnp.int32),
            pltpu.VMEM((_G, _B, ncols), jnp.float32),
            pltpu.VMEM((_G, _B, ncols), jnp.float32),
            pltpu.VMEM((_G, _B, ncols), jnp.float32),
            pltpu.VMEM_SHARED((R, ncols), jnp.float32),
            pltpu.SemaphoreType.DMA,
            pltpu.SemaphoreType.DMA,
            pltpu.SemaphoreType.DMA,
            pltpu.SemaphoreType.DMA,
            pltpu.SemaphoreType.DMA,
            pltpu.SemaphoreType.DMA,
        ],
        compiler_params=pltpu.CompilerParams(use_tc_tiling_on_sc=False),
    )


def _bn_apply(zb, s1_ref, s2_ref, co_ref, g_ref, be_ref, step, N):
    @pl.when(step == 0)
    def _():
        mu = s1_ref[...] / N
        var = s2_ref[...] / N - mu * mu
        scale = g_ref[...] * jax.lax.rsqrt(var + _BN_EPS)
        co_ref[0:1] = scale
        co_ref[1:2] = be_ref[...] - mu * scale

    return jnp.maximum(zb * co_ref[0:1] + co_ref[1:2], 0.0)


def _layer0_tc(agg, x16, wl, wr, b_row, g_row, be_row, wrn, bn_row, N, bn):
    grid = N // bn
    H = wl.shape[1]

    def body(agg_ref, x_ref, wl_ref, wr_ref, b_ref, g_ref, be_ref,
             wrn_ref, bn_ref, h2_ref, hwr_ref, rec_ref,
             zbuf, recbuf, s1_ref, s2_ref, co_ref):
        ph = pl.program_id(0)
        step = pl.program_id(1)

        @pl.when(ph == 0)
        def _():
            a = agg_ref[0] + agg_ref[1]
            xb = x_ref[...]
            rec = 1.0 / (a[:, 15:16] + 1.0)
            mean = (a + xb) * rec
            z = (jnp.dot(mean, wl_ref[...],
                         preferred_element_type=jnp.float32)
                 + jnp.dot(xb, wr_ref[...],
                           preferred_element_type=jnp.float32)
                 + b_ref[...])
            zbuf[pl.ds(step * bn, bn), :] = z
            recbuf[pl.ds(step * bn, bn), :] = rec
            s1 = jnp.sum(z, axis=0, keepdims=True)
            s2 = jnp.sum(z * z, axis=0, keepdims=True)

            @pl.when(step == 0)
            def _():
                s1_ref[...] = s1
                s2_ref[...] = s2

            @pl.when(step != 0)
            def _():
                s1_ref[...] += s1
                s2_ref[...] += s2

        @pl.when(ph == 1)
        def _():
            h = _bn_apply(zbuf[pl.ds(step * bn, bn), :], s1_ref, s2_ref,
                          co_ref, g_ref, be_ref, step, N)
            h2_ref[0] = h[:, :32]
            h2_ref[1] = h[:, 32:]
            hwr_ref[...] = (jnp.dot(h, wrn_ref[...],
                                    preferred_element_type=jnp.float32)
                            + bn_ref[...])
            rec_ref[...] = recbuf[pl.ds(step * bn, bn), :]

    return pl.pallas_call(
        body,
        grid=(2, grid),
        in_specs=[
            pl.BlockSpec((2, bn, 16), lambda p, i: (0, i * (1 - p), 0)),
            pl.BlockSpec((bn, 16), lambda p, i: (i * (1 - p), 0)),
            pl.BlockSpec((16, H), lambda p, i: (0, 0)),
            pl.BlockSpec((16, H), lambda p, i: (0, 0)),
            pl.BlockSpec((1, H), lambda p, i: (0, 0)),
            pl.BlockSpec((1, H), lambda p, i: (0, 0)),
            pl.BlockSpec((1, H), lambda p, i: (0, 0)),
            pl.BlockSpec((H, H), lambda p, i: (0, 0)),
            pl.BlockSpec((1, H), lambda p, i: (0, 0)),
        ],
        out_specs=[
            pl.BlockSpec((2, bn, 32), lambda p, i: (0, i * p, 0)),
            pl.BlockSpec((bn, H), lambda p, i: (i * p, 0)),
            pl.BlockSpec((bn, 1), lambda p, i: (i * p, 0)),
        ],
        out_shape=[
            jax.ShapeDtypeStruct((2, N, 32), jnp.float32),
            jax.ShapeDtypeStruct((N, H), jnp.float32),
            jax.ShapeDtypeStruct((N, 1), jnp.float32),
        ],
        scratch_shapes=[
            pltpu.VMEM((N, H), jnp.float32),
            pltpu.VMEM((N, 1), jnp.float32),
            pltpu.VMEM((1, H), jnp.float32),
            pltpu.VMEM((1, H), jnp.float32),
            pltpu.VMEM((2, H), jnp.float32),
        ],
    )(agg, x16, wl, wr, b_row, g_row, be_row, wrn, bn_row)


def _layer_tc(agg, h2, rec, hwr, wl2, g_row, be_row, wrn, bn_row, N, bn):
    grid = N // bn
    H = hwr.shape[1]

    def body(agg_ref, h_ref, rec_ref, hwr_ref, wl_ref, g_ref, be_ref,
             wrn_ref, bn_ref, h2_ref, hwrn_ref,
             zbuf, s1_ref, s2_ref, co_ref):
        ph = pl.program_id(0)
        step = pl.program_id(1)

        @pl.when(ph == 0)
        def _():
            rec = rec_ref[...]
            ma = (agg_ref[0] + h_ref[0]) * rec
            mb = (agg_ref[1] + h_ref[1]) * rec
            z = (jnp.dot(ma, wl_ref[0], preferred_element_type=jnp.float32)
                 + jnp.dot(mb, wl_ref[1],
                           preferred_element_type=jnp.float32)
                 + hwr_ref[...])
            zbuf[pl.ds(step * bn, bn), :] = z
            s1 = jnp.sum(z, axis=0, keepdims=True)
            s2 = jnp.sum(z * z, axis=0, keepdims=True)

            @pl.when(step == 0)
            def _():
                s1_ref[...] = s1
                s2_ref[...] = s2

            @pl.when(step != 0)
            def _():
                s1_ref[...] += s1
                s2_ref[...] += s2

        @pl.when(ph == 1)
        def _():
            h = _bn_apply(zbuf[pl.ds(step * bn, bn), :], s1_ref, s2_ref,
                          co_ref, g_ref, be_ref, step, N)
            h2_ref[0] = h[:, :32]
            h2_ref[1] = h[:, 32:]
            hwrn_ref[...] = (jnp.dot(h, wrn_ref[...],
                                     preferred_element_type=jnp.float32)
                             + bn_ref[...])

    return pl.pallas_call(
        body,
        grid=(2, grid),
        in_specs=[
            pl.BlockSpec((2, bn, 32), lambda p, i: (0, i * (1 - p), 0)),
            pl.BlockSpec((2, bn, 32), lambda p, i: (0, i * (1 - p), 0)),
            pl.BlockSpec((bn, 1), lambda p, i: (i * (1 - p), 0)),
            pl.BlockSpec((bn, H), lambda p, i: (i * (1 - p), 0)),
            pl.BlockSpec((2, 32, H), lambda p, i: (0, 0, 0)),
            pl.BlockSpec((1, H), lambda p, i: (0, 0)),
            pl.BlockSpec((1, H), lambda p, i: (0, 0)),
            pl.BlockSpec((H, H), lambda p, i: (0, 0)),
            pl.BlockSpec((1, H), lambda p, i: (0, 0)),
        ],
        out_specs=[
            pl.BlockSpec((2, bn, 32), lambda p, i: (0, i * p, 0)),
            pl.BlockSpec((bn, H), lambda p, i: (i * p, 0)),
        ],
        out_shape=[
            jax.ShapeDtypeStruct((2, N, 32), jnp.float32),
            jax.ShapeDtypeStruct((N, H), jnp.float32),
        ],
        scratch_shapes=[
            pltpu.VMEM((N, H), jnp.float32),
            pltpu.VMEM((1, H), jnp.float32),
            pltpu.VMEM((1, H), jnp.float32),
            pltpu.VMEM((2, H), jnp.float32),
        ],
    )(agg, h2, rec, hwr, wl2, g_row, be_row, wrn, bn_row)


def _last_tc(agg, h2, rec, hwr, wl2, g_row, be_row, batchi, wlin, blin11,
             N, NG, bn):
    grid = N // bn
    H = hwr.shape[1]

    def body(agg_ref, h_ref, rec_ref, hwr_ref, wl_ref, g_ref, be_ref,
             b_ref, wlin_ref, blin_ref, o_ref,
             zbuf, s1_ref, s2_ref, co_ref, pool_acc, cnt_acc):
        ph = pl.program_id(0)
        step = pl.program_id(1)

        @pl.when(ph == 0)
        def _():
            rec = rec_ref[...]
            ma = (agg_ref[0] + h_ref[0]) * rec
            mb = (agg_ref[1] + h_ref[1]) * rec
            z = (jnp.dot(ma, wl_ref[0], preferred_element_type=jnp.float32)
                 + jnp.dot(mb, wl_ref[1],
                           preferred_element_type=jnp.float32)
                 + hwr_ref[...])
            zbuf[pl.ds(step * bn, bn), :] = z
            s1 = jnp.sum(z, axis=0, keepdims=True)
            s2 = jnp.sum(z * z, axis=0, keepdims=True)

            @pl.when(step == 0)
            def _():
                s1_ref[...] = s1
                s2_ref[...] = s2

            @pl.when(step != 0)
            def _():
                s1_ref[...] += s1
                s2_ref[...] += s2

        @pl.when(ph == 1)
        def _():
            h = _bn_apply(zbuf[pl.ds(step * bn, bn), :], s1_ref, s2_ref,
                          co_ref, g_ref, be_ref, step, N)
            gids = lax.broadcasted_iota(jnp.int32, (1, NG), 1)
            onehot = (b_ref[...] == gids).astype(jnp.float32)
            psum = lax.dot_general(onehot, h, (((0,), (0,)), ((), ())),
                                   preferred_element_type=jnp.float32)
            ones = jnp.ones((bn, 1), jnp.float32)
            csum = lax.dot_general(onehot, ones, (((0,), (0,)), ((), ())),
                                   preferred_element_type=jnp.float32)

            @pl.when(step == 0)
            def _():
                pool_acc[...] = psum
                cnt_acc[...] = csum

            @pl.when(step != 0)
            def _():
                pool_acc[...] += psum
                cnt_acc[...] += csum

            @pl.when(step == grid - 1)
            def _():
                pooled = pool_acc[...] / jnp.maximum(cnt_acc[...], 1.0)
                o_ref[...] = (jnp.dot(pooled, wlin_ref[...],
                                      preferred_element_type=jnp.float32)
                              + blin_ref[...])

    return pl.pallas_call(
        body,
        grid=(2, grid),
        in_specs=[
            pl.BlockSpec((2, bn, 32), lambda p, i: (0, i * (1 - p), 0)),
            pl.BlockSpec((2, bn, 32), lambda p, i: (0, i * (1 - p), 0)),
            pl.BlockSpec((bn, 1), lambda p, i: (i * (1 - p), 0)),
            pl.BlockSpec((bn, H), lambda p, i: (i * (1 - p), 0)),
            pl.BlockSpec((2, 32, H), lambda p, i: (0, 0, 0)),
            pl.BlockSpec((1, H), lambda p, i: (0, 0)),
            pl.BlockSpec((1, H), lambda p, i: (0, 0)),
            pl.BlockSpec((bn, 1), lambda p, i: (i * p, 0)),
            pl.BlockSpec((H, 1), lambda p, i: (0, 0)),
            pl.BlockSpec((1, 1), lambda p, i: (0, 0)),
        ],
        out_specs=pl.BlockSpec((NG, 1), lambda p, i: (0, 0)),
        out_shape=jax.ShapeDtypeStruct((NG, 1), jnp.float32),
        scratch_shapes=[
            pltpu.VMEM((N, H), jnp.float32),
            pltpu.VMEM((1, H), jnp.float32),
            pltpu.VMEM((1, H), jnp.float32),
            pltpu.VMEM((2, H), jnp.float32),
            pltpu.VMEM((NG, H), jnp.float32),
            pltpu.VMEM((NG, 1), jnp.float32),
        ],
    )(agg, h2, rec, hwr, wl2, g_row, be_row, batchi, wlin, blin11)


def kernel(x, edge_index, batch, y,
           Wl0, Wr0, b0, g0, be0,
           Wl1, Wr1, b1, g1, be1,
           Wl2, Wr2, b2, g2, be2,
           Wl3, Wr3, b3, g3, be3,
           Wl4, Wr4, b4, g4, be4,
           Wlin, blin):
    N, DIN = x.shape
    E = edge_index.shape[1]
    H = Wl0.shape[1]
    NG = y.shape[0]
    NL = 5
    bn = 1000

    chunk = _NC * _NS * _EG
    EP = -(-E // chunk) * chunk
    R = -(-(N + 48) // _NS) * _NS
    P = EP - E

    src = edge_index[0]
    dst = edge_index[1]
    pad_i = jnp.arange(P, dtype=jnp.int32)
    src_p = jnp.concatenate([src, pad_i % jnp.int32(N)])
    dst_p = jnp.concatenate([dst, jnp.int32(N) + pad_i % jnp.int32(R - N)])
    dstm = dst_p.reshape(EP // _B, _B)
    srcm1 = src_p.reshape(1, EP // _B, _B)
    srcm2 = jnp.stack([src_p, src_p + jnp.int32(N)]).reshape(2, EP // _B, _B)

    zeros16 = jnp.zeros((R, 16), jnp.float32)
    zeros32 = jnp.zeros((R, 32), jnp.float32)
    x16 = jnp.concatenate([x, jnp.ones((N, 1), jnp.float32)], axis=1)

    zrow = jnp.zeros((1, H), jnp.float32)
    wl0p = jnp.concatenate([Wl0, zrow], axis=0)
    wr0p = jnp.concatenate([Wr0, zrow], axis=0)

    agg16 = _make_sc_agg(R, 16, EP, split_edges=True, nslabs=1)
    agg32 = _make_sc_agg(R, 32, EP, split_edges=False, nslabs=2)

    a0 = agg16(x16, srcm1, dstm, zeros16)
    h2, hwr, rec = _layer0_tc(a0, x16, wl0p, wr0p, b0.reshape(1, H),
                              g0.reshape(1, H), be0.reshape(1, H),
                              Wr1, b1.reshape(1, H), N, bn)

    layers = [(Wl1, g1, be1, Wr2, b2), (Wl2, g2, be2, Wr3, b3),
              (Wl3, g3, be3, Wr4, b4)]
    for wl, g, be, wrn, bnx in layers:
        table = h2.reshape(2 * N, 32)
        agg = agg32(table, srcm2, dstm, zeros32)
        h2, hwr = _layer_tc(agg, h2, rec, hwr, wl.reshape(2, 32, H),
                            g.reshape(1, H), be.reshape(1, H),
                            wrn, bnx.reshape(1, H), N, bn)

    table = h2.reshape(2 * N, 32)
    agg = agg32(table, srcm2, dstm, zeros32)
    batchi = batch.reshape(N, 1)
    out = _last_tc(agg, h2, rec, hwr, Wl4.reshape(2, 32, H),
                   g4.reshape(1, H), be4.reshape(1, H), batchi,
                   Wlin, blin.reshape(1, 1), N, NG, bn)
    return out
